# baseline (speedup 1.0000x reference)
import jax
import jax.numpy as jnp
from jax import lax
from jax.experimental import pallas as pl
from jax.experimental.pallas import tpu as pltpu
from jax.experimental.pallas import tpu_sc as plsc

F = 16
NC = 2
NS = 16
NW = NC * NS


def _relayout_rows(t):
    n = t.shape[0]
    tt = jnp.swapaxes(t, 0, 1)
    c = min(n, 16384)
    grid = (n + c - 1) // c

    def body(in_ref, out_ref):
        out_ref[...] = in_ref[...].T

    return pl.pallas_call(
        body,
        grid=(grid,),
        in_specs=[pl.BlockSpec((F, c), lambda i: (0, i))],
        out_specs=pl.BlockSpec((c, F), lambda i: (i, 0)),
        out_shape=jax.ShapeDtypeStruct((n, F), jnp.float32),
    )(tt)


def kernel(user, item, metadata, user_table, item_table, meta_table0,
           meta_table1, lin_user, lin_item, lin_meta0, lin_meta1):
    b = user.shape[0]
    assert b % (8 * NW) == 0
    bpw = b // NW

    m0c = metadata[:, 0].astype(jnp.int32)
    m1c = metadata[:, 1].astype(jnp.int32)
    user = user.astype(jnp.int32)
    item = item.astype(jnp.int32)

    user_table = _relayout_rows(user_table)
    item_table = _relayout_rows(item_table)
    meta_table0 = _relayout_rows(meta_table0)
    meta_table1 = _relayout_rows(meta_table1)
    lu_flat = lin_user.reshape(-1)
    li_flat = lin_item.reshape(-1)
    l0_flat = lin_meta0.reshape(-1)
    l1_flat = lin_meta1.reshape(-1)

    mesh = plsc.VectorSubcoreMesh(
        core_axis_name="c", subcore_axis_name="s",
        num_cores=NC, num_subcores=NS)

    def body(user_hbm, item_hbm, m0_hbm, m1_hbm,
             ut_hbm, it_hbm, t0_hbm, t1_hbm,
             lu_hbm, li_hbm, l0_hbm, l1_hbm,
             out_hbm,
             uidx, iidx, m0idx, m1idx,
             urows, irows, arows, brows,
             lu, li, l0, l1,
             pwbuf, outbuf, sem):
        wid = lax.axis_index("s") * NC + lax.axis_index("c")
        base = wid * bpw

        pltpu.sync_copy(user_hbm.at[pl.ds(base, bpw)], uidx)
        pltpu.sync_copy(item_hbm.at[pl.ds(base, bpw)], iidx)
        pltpu.sync_copy(m0_hbm.at[pl.ds(base, bpw)], m0idx)
        pltpu.sync_copy(m1_hbm.at[pl.ds(base, bpw)], m1idx)

        cps = [
            pltpu.async_copy(ut_hbm.at[uidx], urows, sem),
            pltpu.async_copy(it_hbm.at[iidx], irows, sem),
            pltpu.async_copy(t0_hbm.at[m0idx], arows, sem),
            pltpu.async_copy(t1_hbm.at[m1idx], brows, sem),
            pltpu.async_copy(lu_hbm.at[uidx], lu, sem),
            pltpu.async_copy(li_hbm.at[iidx], li, sem),
            pltpu.async_copy(l0_hbm.at[m0idx], l0, sem),
            pltpu.async_copy(l1_hbm.at[m1idx], l1, sem),
        ]
        for cp in cps:
            cp.wait()

        last_lane = lax.iota(jnp.int32, F) == (F - 1)

        @plsc.parallel_loop(0, bpw, unroll=8)
        def _(j):
            u = urows[j]
            it = irows[j]
            a = arows[j]
            c = brows[j]
            s = u + it + a + c
            q = s * s - u * u - it * it - a * a - c * c
            cs = plsc.cumsum(q)
            plsc.store_scatter(pwbuf, [jnp.full((F,), j, jnp.int32)], cs,
                               mask=last_lane)

        for g in range(bpw // F):
            sl = pl.ds(g * F, F)
            x = lu[sl] + li[sl] + l0[sl] + l1[sl] + 0.5 * pwbuf[sl]
            outbuf[sl] = 1.0 / (1.0 + jnp.exp(-x))

        pltpu.sync_copy(outbuf, out_hbm.at[pl.ds(base, bpw)])

    fm = pl.kernel(
        body,
        out_type=jax.ShapeDtypeStruct((b,), jnp.float32),
        mesh=mesh,
        compiler_params=pltpu.CompilerParams(
            needs_layout_passes=False, use_tc_tiling_on_sc=False),
        scratch_types=[
            pltpu.VMEM((bpw,), jnp.int32),
            pltpu.VMEM((bpw,), jnp.int32),
            pltpu.VMEM((bpw,), jnp.int32),
            pltpu.VMEM((bpw,), jnp.int32),
            pltpu.VMEM((bpw, F), jnp.float32),
            pltpu.VMEM((bpw, F), jnp.float32),
            pltpu.VMEM((bpw, F), jnp.float32),
            pltpu.VMEM((bpw, F), jnp.float32),
            pltpu.VMEM((bpw,), jnp.float32),
            pltpu.VMEM((bpw,), jnp.float32),
            pltpu.VMEM((bpw,), jnp.float32),
            pltpu.VMEM((bpw,), jnp.float32),
            pltpu.VMEM((bpw,), jnp.float32),
            pltpu.VMEM((bpw,), jnp.float32),
            pltpu.SemaphoreType.DMA,
        ],
    )
    return fm(user, item, m0c, m1c,
              user_table, item_table, meta_table0, meta_table1,
              lu_flat, li_flat, l0_flat, l1_flat)


# speedup vs baseline: 1.2383x; 1.2383x over previous
import functools

import jax
import jax.numpy as jnp
from jax import lax
from jax.experimental import pallas as pl
from jax.experimental.pallas import tpu as pltpu
from jax.experimental.pallas import tpu_sc as plsc

F = 16
NC = 2
NS = 16
NW = NC * NS
S = 16
LANE = 128
NMETA = 1000


def _fetch_tables(ut_t, it_t, utile, ucol, itile, icol, b):
    grid = b // S

    def body(utile_ref, ucol_ref, itile_ref, icol_ref, *refs):
        ublocks = refs[:S]
        iblocks = refs[S:2 * S]
        uout, iout = refs[2 * S], refs[2 * S + 1]
        step = pl.program_id(0)
        col_iota = lax.broadcasted_iota(jnp.int32, (F, LANE), 1)

        def extract(blk, c):
            m = (col_iota == c).astype(jnp.float32)
            return jnp.sum(blk * m, axis=1, keepdims=True)

        ucols = []
        icols = []
        for k in range(S):
            cu = ucol_ref[step * S + k]
            ci = icol_ref[step * S + k]
            ucols.append(extract(ublocks[k][...], cu))
            icols.append(extract(iblocks[k][...], ci))
        uout[...] = jnp.concatenate(ucols, axis=1).reshape(1, F, S)
        iout[...] = jnp.concatenate(icols, axis=1).reshape(1, F, S)

    def u_map(k):
        return lambda i, ut, uc, itl, ic: (0, ut[i * S + k])

    def i_map(k):
        return lambda i, ut, uc, itl, ic: (0, itl[i * S + k])

    grid_spec = pltpu.PrefetchScalarGridSpec(
        num_scalar_prefetch=4,
        grid=(grid,),
        in_specs=(
            [pl.BlockSpec((F, LANE), u_map(k)) for k in range(S)]
            + [pl.BlockSpec((F, LANE), i_map(k)) for k in range(S)]
        ),
        out_specs=[
            pl.BlockSpec((1, F, S), lambda i, *_: (i, 0, 0)),
            pl.BlockSpec((1, F, S), lambda i, *_: (i, 0, 0)),
        ],
    )
    return pl.pallas_call(
        body,
        grid_spec=grid_spec,
        out_shape=[
            jax.ShapeDtypeStruct((grid, F, S), jnp.float32),
            jax.ShapeDtypeStruct((grid, F, S), jnp.float32),
        ],
    )(utile, ucol, itile, icol, *([ut_t] * S), *([it_t] * S))


def kernel(user, item, metadata, user_table, item_table, meta_table0,
           meta_table1, lin_user, lin_item, lin_meta0, lin_meta1):
    b = user.shape[0]
    assert b % (S * NW) == 0
    bpw = b // NW
    gpw = bpw // S

    user = user.astype(jnp.int32)
    item = item.astype(jnp.int32)
    m0c = metadata[:, 0].astype(jnp.int32)
    m1c = metadata[:, 1].astype(jnp.int32)

    ut_t = jnp.swapaxes(user_table, 0, 1)
    it_t = jnp.swapaxes(item_table, 0, 1)
    utile, ucol = user // LANE, user % LANE
    itile, icol = item // LANE, item % LANE

    uemb, iemb = _fetch_tables(ut_t, it_t, utile, ucol, itile, icol, b)

    m0_t = jnp.swapaxes(meta_table0[:NMETA], 0, 1)
    m1_t = jnp.swapaxes(meta_table1[:NMETA], 0, 1)
    lm0 = lin_meta0[:NMETA].reshape(-1)
    lm1 = lin_meta1[:NMETA].reshape(-1)
    lu = lin_user.reshape(-1)
    li = lin_item.reshape(-1)

    mesh = plsc.VectorSubcoreMesh(
        core_axis_name="c", subcore_axis_name="s",
        num_cores=NC, num_subcores=NS)

    def body(uemb_hbm, iemb_hbm, m0t_hbm, m1t_hbm, lm0_hbm, lm1_hbm,
             lu_hbm, li_hbm, uidx_hbm, iidx_hbm, m0_hbm, m1_hbm,
             out_hbm,
             ubuf, ibuf, m0tab, m1tab, lm0buf, lm1buf,
             uidxb, iidxb, m0ib, m1ib, lubuf, libuf, outbuf, sem):
        wid = lax.axis_index("s") * NC + lax.axis_index("c")
        base = wid * bpw
        g0 = wid * gpw

        pltpu.sync_copy(uemb_hbm.at[pl.ds(g0, gpw)], ubuf)
        pltpu.sync_copy(iemb_hbm.at[pl.ds(g0, gpw)], ibuf)
        pltpu.sync_copy(m0t_hbm, m0tab)
        pltpu.sync_copy(m1t_hbm, m1tab)
        pltpu.sync_copy(lm0_hbm, lm0buf)
        pltpu.sync_copy(lm1_hbm, lm1buf)
        pltpu.sync_copy(uidx_hbm.at[pl.ds(base, bpw)], uidxb)
        pltpu.sync_copy(iidx_hbm.at[pl.ds(base, bpw)], iidxb)
        pltpu.sync_copy(m0_hbm.at[pl.ds(base, bpw)], m0ib)
        pltpu.sync_copy(m1_hbm.at[pl.ds(base, bpw)], m1ib)

        cps = [
            pltpu.async_copy(lu_hbm.at[uidxb], lubuf, sem),
            pltpu.async_copy(li_hbm.at[iidxb], libuf, sem),
        ]
        for cp in cps:
            cp.wait()

        @plsc.parallel_loop(0, gpw, unroll=2)
        def _(g):
            sl = pl.ds(g * F, F)
            mi0 = m0ib[sl]
            mi1 = m1ib[sl]
            pw = jnp.zeros((F,), jnp.float32)
            for f in range(F):
                fv = jnp.full((F,), f, jnp.int32)
                u = ubuf[g, f]
                it = ibuf[g, f]
                a = plsc.load_gather(m0tab, [fv, mi0])
                c2 = plsc.load_gather(m1tab, [fv, mi1])
                s = u + it + a + c2
                pw = pw + (s * s - (u * u + it * it + a * a + c2 * c2))
            lin = (lubuf[sl] + libuf[sl]
                   + plsc.load_gather(lm0buf, [mi0])
                   + plsc.load_gather(lm1buf, [mi1]))
            x = lin + 0.5 * pw
            outbuf[sl] = 1.0 / (1.0 + jnp.exp(-x))

        pltpu.sync_copy(outbuf, out_hbm.at[pl.ds(base, bpw)])

    fm = pl.kernel(
        body,
        out_type=jax.ShapeDtypeStruct((b,), jnp.float32),
        mesh=mesh,
        compiler_params=pltpu.CompilerParams(
            needs_layout_passes=False, use_tc_tiling_on_sc=False),
        scratch_types=[
            pltpu.VMEM((gpw, F, S), jnp.float32),
            pltpu.VMEM((gpw, F, S), jnp.float32),
            pltpu.VMEM((F, NMETA), jnp.float32),
            pltpu.VMEM((F, NMETA), jnp.float32),
            pltpu.VMEM((NMETA,), jnp.float32),
            pltpu.VMEM((NMETA,), jnp.float32),
            pltpu.VMEM((bpw,), jnp.int32),
            pltpu.VMEM((bpw,), jnp.int32),
            pltpu.VMEM((bpw,), jnp.int32),
            pltpu.VMEM((bpw,), jnp.int32),
            pltpu.VMEM((bpw,), jnp.float32),
            pltpu.VMEM((bpw,), jnp.float32),
            pltpu.VMEM((bpw,), jnp.float32),
            pltpu.SemaphoreType.DMA,
        ],
    )
    return fm(uemb, iemb, m0_t, m1_t, lm0, lm1, lu, li,
              user, item, m0c, m1c)


# speedup vs baseline: 4.0801x; 3.2949x over previous
import jax
import jax.numpy as jnp
from jax import lax
from jax.experimental import pallas as pl
from jax.experimental.pallas import tpu as pltpu
from jax.experimental.pallas import tpu_sc as plsc

F = 16
NC = 2
NS = 16
NW = NC * NS
LANE = 128
NMAIN = 999424
NBLK = NMAIN // LANE
KSTEP = 64
GRID = NBLK // KSTEP
NTAIL = 1000000 - NMAIN
NMETA = 1000


def _detile(ut_t, it_t):
    def body(u_ref, i_ref, uo_ref, io_ref):
        uo_ref[...] = u_ref[...].reshape(F, KSTEP, LANE)
        io_ref[...] = i_ref[...].reshape(F, KSTEP, LANE)

    return pl.pallas_call(
        body,
        grid=(GRID,),
        in_specs=[
            pl.BlockSpec((F, KSTEP * LANE), lambda s: (0, s)),
            pl.BlockSpec((F, KSTEP * LANE), lambda s: (0, s)),
        ],
        out_specs=[
            pl.BlockSpec((F, KSTEP, LANE), lambda s: (0, s, 0)),
            pl.BlockSpec((F, KSTEP, LANE), lambda s: (0, s, 0)),
        ],
        out_shape=[
            jax.ShapeDtypeStruct((F, NBLK, LANE), jnp.float32),
            jax.ShapeDtypeStruct((F, NBLK, LANE), jnp.float32),
        ],
    )(ut_t, it_t)


def kernel(user, item, metadata, user_table, item_table, meta_table0,
           meta_table1, lin_user, lin_item, lin_meta0, lin_meta1):
    b = user.shape[0]
    assert b % (F * NW) == 0
    bpw = b // NW
    gpw = bpw // F

    user = user.astype(jnp.int32)
    item = item.astype(jnp.int32)
    m0c = metadata[:, 0].astype(jnp.int32)
    m1c = metadata[:, 1].astype(jnp.int32)

    ut_t = jnp.swapaxes(user_table, 0, 1)
    it_t = jnp.swapaxes(item_table, 0, 1)
    uflat, iflat = _detile(ut_t, it_t)
    uflat = uflat.reshape(-1)
    iflat = iflat.reshape(-1)

    utail = jnp.swapaxes(user_table[NMAIN:], 0, 1)
    itail = jnp.swapaxes(item_table[NMAIN:], 0, 1)
    m0_t = jnp.swapaxes(meta_table0[:NMETA], 0, 1)
    m1_t = jnp.swapaxes(meta_table1[:NMETA], 0, 1)
    lm0 = lin_meta0[:NMETA].reshape(-1)
    lm1 = lin_meta1[:NMETA].reshape(-1)
    lu = lin_user.reshape(-1)
    li = lin_item.reshape(-1)

    mesh = plsc.VectorSubcoreMesh(
        core_axis_name="c", subcore_axis_name="s",
        num_cores=NC, num_subcores=NS)

    def body(uflat_hbm, iflat_hbm, utail_hbm, itail_hbm,
             m0t_hbm, m1t_hbm, lm0_hbm, lm1_hbm, lu_hbm, li_hbm,
             uidx_hbm, iidx_hbm, m0_hbm, m1_hbm,
             out_hbm,
             utailb, itailb, m0tab, m1tab, lm0buf, lm1buf,
             uidxb, iidxb, m0ib, m1ib,
             uoff, ioff, uemb, iemb, lubuf, libuf, outbuf, sem):
        wid = lax.axis_index("s") * NC + lax.axis_index("c")
        base = wid * bpw

        pltpu.sync_copy(utail_hbm, utailb)
        pltpu.sync_copy(itail_hbm, itailb)
        pltpu.sync_copy(m0t_hbm, m0tab)
        pltpu.sync_copy(m1t_hbm, m1tab)
        pltpu.sync_copy(lm0_hbm, lm0buf)
        pltpu.sync_copy(lm1_hbm, lm1buf)
        pltpu.sync_copy(uidx_hbm.at[pl.ds(base, bpw)], uidxb)
        pltpu.sync_copy(iidx_hbm.at[pl.ds(base, bpw)], iidxb)
        pltpu.sync_copy(m0_hbm.at[pl.ds(base, bpw)], m0ib)
        pltpu.sync_copy(m1_hbm.at[pl.ds(base, bpw)], m1ib)

        @plsc.parallel_loop(0, gpw, unroll=2)
        def _(g):
            sl = pl.ds(g * F, F)
            ui = jnp.minimum(uidxb[sl], NMAIN - 1)
            ii = jnp.minimum(iidxb[sl], NMAIN - 1)
            for f in range(F):
                uoff[pl.ds(f * bpw + g * F, F)] = ui + f * NMAIN
                ioff[pl.ds(f * bpw + g * F, F)] = ii + f * NMAIN

        cps = [
            pltpu.async_copy(uflat_hbm.at[uoff], uemb, sem),
            pltpu.async_copy(iflat_hbm.at[ioff], iemb, sem),
            pltpu.async_copy(lu_hbm.at[uidxb], lubuf, sem),
            pltpu.async_copy(li_hbm.at[iidxb], libuf, sem),
        ]
        for cp in cps:
            cp.wait()

        @plsc.parallel_loop(0, gpw, unroll=2)
        def _(g):
            sl = pl.ds(g * F, F)
            uidx = uidxb[sl]
            iidx = iidxb[sl]
            mi0 = m0ib[sl]
            mi1 = m1ib[sl]
            utailm = uidx >= NMAIN
            itailm = iidx >= NMAIN
            uti = jnp.maximum(uidx - NMAIN, 0)
            iti = jnp.maximum(iidx - NMAIN, 0)
            pw = jnp.zeros((F,), jnp.float32)
            for f in range(F):
                fv = jnp.full((F,), f, jnp.int32)
                u = uemb[pl.ds(f * bpw + g * F, F)]
                it = iemb[pl.ds(f * bpw + g * F, F)]
                u = jnp.where(utailm, plsc.load_gather(utailb, [fv, uti]), u)
                it = jnp.where(itailm, plsc.load_gather(itailb, [fv, iti]), it)
                a = plsc.load_gather(m0tab, [fv, mi0])
                c2 = plsc.load_gather(m1tab, [fv, mi1])
                s = u + it + a + c2
                pw = pw + (s * s - (u * u + it * it + a * a + c2 * c2))
            lin = (lubuf[sl] + libuf[sl]
                   + plsc.load_gather(lm0buf, [mi0])
                   + plsc.load_gather(lm1buf, [mi1]))
            x = lin + 0.5 * pw
            outbuf[sl] = 1.0 / (1.0 + jnp.exp(-x))

        pltpu.sync_copy(outbuf, out_hbm.at[pl.ds(base, bpw)])

    fm = pl.kernel(
        body,
        out_type=jax.ShapeDtypeStruct((b,), jnp.float32),
        mesh=mesh,
        compiler_params=pltpu.CompilerParams(
            needs_layout_passes=False, use_tc_tiling_on_sc=False),
        scratch_types=[
            pltpu.VMEM((F, NTAIL), jnp.float32),
            pltpu.VMEM((F, NTAIL), jnp.float32),
            pltpu.VMEM((F, NMETA), jnp.float32),
            pltpu.VMEM((F, NMETA), jnp.float32),
            pltpu.VMEM((NMETA,), jnp.float32),
            pltpu.VMEM((NMETA,), jnp.float32),
            pltpu.VMEM((bpw,), jnp.int32),
            pltpu.VMEM((bpw,), jnp.int32),
            pltpu.VMEM((bpw,), jnp.int32),
            pltpu.VMEM((bpw,), jnp.int32),
            pltpu.VMEM((F * bpw,), jnp.int32),
            pltpu.VMEM((F * bpw,), jnp.int32),
            pltpu.VMEM((F * bpw,), jnp.float32),
            pltpu.VMEM((F * bpw,), jnp.float32),
            pltpu.VMEM((bpw,), jnp.float32),
            pltpu.VMEM((bpw,), jnp.float32),
            pltpu.VMEM((bpw,), jnp.float32),
            pltpu.SemaphoreType.DMA,
        ],
    )
    return fm(uflat, iflat, utail, itail, m0_t, m1_t, lm0, lm1, lu, li,
              user, item, m0c, m1c)
